# baseline (speedup 1.0000x reference)
import functools

import jax
import jax.numpy as jnp
from jax import lax
from jax.experimental import pallas as pl
from jax.experimental.pallas import tpu as pltpu
from jax.experimental.pallas import tpu_sc as plsc

N0 = 5000
N1 = 5000
N = N0 + N1
E = 320000
ER = E // 2
Q = 100000
IN = 128
HID = 128
OUT = 64

NC = 2
NS = 16
NW = NC * NS
K = 128
TRIPS = 80
HTRIPS = TRIPS // 2
CE = TRIPS * K
ERP = CE * NS
EP = 2 * ERP
U = 2
GROUPS = HTRIPS // U
CHUNK = 640
TAIL = N - (NS - 1) * CHUNK
CNT_CHUNK = 640
QP = 102400
QT = QP // NW
KD = 128
DTRIPS = QT // KD

_f32 = jnp.float32
_i32 = jnp.int32


def _m1_body(x0_ref, x1_ref, lw0_ref, lb0_ref, lw1_ref, lb1_ref, wrel_ref,
             wroot_ref, b1_ref, t_ref, root_ref):
    h0 = jnp.dot(x0_ref[...], lw0_ref[...], preferred_element_type=_f32) + lb0_ref[...]
    h1 = jnp.dot(x1_ref[...], lw1_ref[...], preferred_element_type=_f32) + lb1_ref[...]
    h = jnp.concatenate([h0, h1], axis=0)
    t_ref[...] = jnp.concatenate(
        [jnp.dot(h, wrel_ref[0], preferred_element_type=_f32),
         jnp.dot(h, wrel_ref[1], preferred_element_type=_f32)], axis=0)
    root_ref[...] = jnp.dot(h, wroot_ref[...], preferred_element_type=_f32) + b1_ref[...]


_m1 = pl.pallas_call(
    _m1_body,
    out_shape=[
        jax.ShapeDtypeStruct((2 * N, HID), _f32),
        jax.ShapeDtypeStruct((N, HID), _f32),
    ],
)


def _m2_body(root_ref, a0_ref, a1_ref, c0_ref, c1_ref, wrel_ref, wroot_ref,
             b_ref, tp_ref, root2_ref):
    inv0 = 1.0 / jnp.maximum(c0_ref[...], 1.0)
    inv1 = 1.0 / jnp.maximum(c1_ref[...], 1.0)
    z = root_ref[...] + a0_ref[...] * inv0[:, None] + a1_ref[...] * inv1[:, None]
    z = jnp.maximum(z, 0.0)
    tp_ref[...] = jnp.concatenate(
        [jnp.dot(z, wrel_ref[0], preferred_element_type=_f32),
         jnp.dot(z, wrel_ref[1], preferred_element_type=_f32)], axis=1)
    root2_ref[...] = jnp.dot(z, wroot_ref[...], preferred_element_type=_f32) + b_ref[...]


_m2 = pl.pallas_call(
    _m2_body,
    out_shape=[
        jax.ShapeDtypeStruct((N, 2 * OUT), _f32),
        jax.ShapeDtypeStruct((N, OUT), _f32),
    ],
)


def _m3_body(root_ref, a0_ref, a1_ref, c0_ref, c1_ref, wuv_ref, buv_ref, uv_ref):
    inv0 = 1.0 / jnp.maximum(c0_ref[...], 1.0)
    inv1 = 1.0 / jnp.maximum(c1_ref[...], 1.0)
    a0 = a0_ref[...][:, :OUT]
    a1 = a1_ref[...][:, OUT:]
    z = root_ref[...] + a0 * inv0[:, None] + a1 * inv1[:, None]
    uv_ref[...] = jnp.dot(z, wuv_ref[...], preferred_element_type=_f32) + buv_ref[...]


_m3 = pl.pallas_call(
    _m3_body,
    out_shape=jax.ShapeDtypeStruct((N, 2), _f32),
)


def _fill_vec(ref, n, value):
    def body(j, carry):
        ref[pl.ds(j * 16, 16)] = jnp.full((16,), value, _f32)
        return carry
    lax.fori_loop(0, n // 16, body, 0)


def _make_agg(d, with_counts):
    mesh = plsc.VectorSubcoreMesh(
        core_axis_name="c", subcore_axis_name="s", num_cores=NC, num_subcores=NS)
    out_type = [
        jax.ShapeDtypeStruct((N, d), _f32),
        jax.ShapeDtypeStruct((N, d), _f32),
    ]
    scratch = [
        pltpu.VMEM((HTRIPS, K), _i32),
        pltpu.VMEM((HTRIPS, K), _i32),
        pltpu.VMEM((U, K, d), _f32),
        pltpu.VMEM_SHARED((N + 8, d), _f32),
        pltpu.SemaphoreType.DMA((U,)),
        pltpu.SemaphoreType.DMA((U,)),
    ]
    if with_counts:
        out_type += [
            jax.ShapeDtypeStruct((N,), _f32),
            jax.ShapeDtypeStruct((N,), _f32),
        ]
        scratch += [
            pltpu.VMEM((K,), _f32),
            pltpu.VMEM((CNT_CHUNK,), _f32),
            pltpu.VMEM_SHARED((N + 8,), _f32),
            pltpu.SemaphoreType.DMA((U,)),
        ]

    def body(table_hbm, srcp2_hbm, dst2_hbm, zeros_hbm, agg0_out, agg1_out, *rest):
        if with_counts:
            (cnt0_out, cnt1_out, sidx_v, didx_v, rows_v, acc_sh,
             gsem, ssem, ones_v, zcnt_v, cnt_sh, csem) = rest
        else:
            sidx_v, didx_v, rows_v, acc_sh, gsem, ssem = rest
        c = lax.axis_index("c")
        s = lax.axis_index("s")
        idx_row0 = c * (ERP // K) + s * TRIPS

        @pl.when(s < NS - 1)
        def _():
            pltpu.sync_copy(zeros_hbm, acc_sh.at[pl.ds(s * CHUNK, CHUNK)])

        @pl.when(s == NS - 1)
        def _():
            pltpu.sync_copy(zeros_hbm.at[pl.ds(0, TAIL)],
                            acc_sh.at[pl.ds((NS - 1) * CHUNK, TAIL)])

        if with_counts:
            _fill_vec(ones_v, K, 1.0)
            _fill_vec(zcnt_v, CNT_CHUNK, 0.0)

            @pl.when(s < NS - 1)
            def _():
                pltpu.sync_copy(zcnt_v, cnt_sh.at[pl.ds(s * CNT_CHUNK, CNT_CHUNK)])

            @pl.when(s == NS - 1)
            def _():
                pltpu.sync_copy(zcnt_v.at[pl.ds(0, N - (NS - 1) * CNT_CHUNK)],
                                cnt_sh.at[pl.ds((NS - 1) * CNT_CHUNK,
                                                N - (NS - 1) * CNT_CHUNK)])
        plsc.subcore_barrier()

        def group(g, carry):
            gd, sd, cd = [], [], []
            for j in range(U):
                t = g * U + j
                gd.append(pltpu.async_copy(
                    table_hbm.at[sidx_v.at[t]], rows_v.at[j], gsem.at[j]))
            for j in range(U):
                t = g * U + j
                gd[j].wait()
                sd.append(pltpu.async_copy(
                    rows_v.at[j], acc_sh.at[didx_v.at[t]], ssem.at[j], add=True))
                if with_counts:
                    cd.append(pltpu.async_copy(
                        ones_v, cnt_sh.at[didx_v.at[t]], csem.at[j], add=True))
            for j in range(U):
                sd[j].wait()
                if with_counts:
                    cd[j].wait()
            return carry

        for half in range(2):
            pltpu.sync_copy(srcp2_hbm.at[pl.ds(idx_row0 + half * HTRIPS, HTRIPS)],
                            sidx_v)
            pltpu.sync_copy(dst2_hbm.at[pl.ds(idx_row0 + half * HTRIPS, HTRIPS)],
                            didx_v)
            lax.fori_loop(0, GROUPS, group, 0)
        plsc.subcore_barrier()

        for cc, agg_out in ((0, agg0_out), (1, agg1_out)):
            @pl.when(jnp.logical_and(c == cc, s < NS - 1))
            def _(agg_out=agg_out):
                pltpu.sync_copy(acc_sh.at[pl.ds(s * CHUNK, CHUNK)],
                                agg_out.at[pl.ds(s * CHUNK, CHUNK)])

            @pl.when(jnp.logical_and(c == cc, s == NS - 1))
            def _(agg_out=agg_out):
                pltpu.sync_copy(acc_sh.at[pl.ds((NS - 1) * CHUNK, TAIL)],
                                agg_out.at[pl.ds((NS - 1) * CHUNK, TAIL)])

        if with_counts:
            tail = N - (NS - 1) * CNT_CHUNK
            for cc, cnt_out in ((0, cnt0_out), (1, cnt1_out)):
                @pl.when(jnp.logical_and(c == cc, s < NS - 1))
                def _(cnt_out=cnt_out):
                    pltpu.sync_copy(cnt_sh.at[pl.ds(s * CNT_CHUNK, CNT_CHUNK)], zcnt_v)
                    pltpu.sync_copy(zcnt_v, cnt_out.at[pl.ds(s * CNT_CHUNK, CNT_CHUNK)])

                @pl.when(jnp.logical_and(c == cc, s == NS - 1))
                def _(cnt_out=cnt_out):
                    pltpu.sync_copy(cnt_sh.at[pl.ds((NS - 1) * CNT_CHUNK, tail)],
                                    zcnt_v.at[pl.ds(0, tail)])
                    pltpu.sync_copy(zcnt_v.at[pl.ds(0, tail)],
                                    cnt_out.at[pl.ds((NS - 1) * CNT_CHUNK, tail)])

    return pl.kernel(body, out_type=out_type, mesh=mesh, scratch_types=scratch)


_agg_cache = functools.lru_cache(maxsize=None)(_make_agg)


def _make_decode():
    mesh = plsc.VectorSubcoreMesh(
        core_axis_name="c", subcore_axis_name="s", num_cores=NC, num_subcores=NS)
    out_type = jax.ShapeDtypeStruct((QP,), _f32)
    scratch = [
        pltpu.VMEM((N,), _f32),
        pltpu.VMEM((N,), _f32),
        pltpu.VMEM((QT,), _i32),
        pltpu.VMEM((QT,), _i32),
        pltpu.VMEM((KD,), _f32),
    ]

    def body(u_hbm, v_hbm, i0_hbm, i1_hbm, out_hbm, u_v, v_v, i0_v, i1_v, r_v):
        c = lax.axis_index("c")
        s = lax.axis_index("s")
        w = s * NC + c
        base = w * QT
        pltpu.sync_copy(u_hbm, u_v)
        pltpu.sync_copy(v_hbm, v_v)
        pltpu.sync_copy(i0_hbm.at[pl.ds(base, QT)], i0_v)
        pltpu.sync_copy(i1_hbm.at[pl.ds(base, QT)], i1_v)

        def step(i, carry):
            for j in range(KD // 16):
                a = plsc.load_gather(u_v, [i0_v[pl.ds(i * KD + j * 16, 16)]])
                b = plsc.load_gather(v_v, [i1_v[pl.ds(i * KD + j * 16, 16)]])
                x = a + b
                r_v[pl.ds(j * 16, 16)] = 1.0 / (1.0 + jnp.exp(-x))
            off = pl.multiple_of(base + i * KD, 8)
            pltpu.sync_copy(r_v, out_hbm.at[pl.ds(off, KD)])
            return carry

        lax.fori_loop(0, DTRIPS, step, 0)

    return pl.kernel(body, out_type=out_type, mesh=mesh, scratch_types=scratch,
                     compiler_params=pltpu.CompilerParams(
                         use_tc_tiling_on_sc=False, needs_layout_passes=False))


_decode_cache = functools.lru_cache(maxsize=None)(_make_decode)


def kernel(x0, x1, edge_index, index, lin0_W, lin0_b, lin1_W, lin1_b,
           W1_rel, W1_root, b1, W2_rel, W2_root, b2, fc_W, fc_b):
    src = jnp.asarray(edge_index[0], _i32)
    dst = jnp.asarray(edge_index[1], _i32)
    i0 = jnp.asarray(index[0], _i32)
    i1 = jnp.asarray(index[1], _i32)

    npad = ERP - ER
    pad0 = jnp.zeros((npad,), _i32)
    padN = jnp.full((npad,), N, _i32)
    srcp1 = jnp.concatenate([src[:ER], pad0, src[ER:] + N, pad0]).reshape(EP // K, K)
    srcp2 = jnp.concatenate([src[:ER], pad0, src[ER:], pad0]).reshape(EP // K, K)
    dst2 = jnp.concatenate([dst[:ER], padN, dst[ER:], padN]).reshape(EP // K, K)
    i0p = jnp.concatenate([i0, jnp.zeros((QP - Q,), _i32)])
    i1p = jnp.concatenate([i1, jnp.zeros((QP - Q,), _i32)])

    t1, root1 = _m1(
        x0, x1, lin0_W, lin0_b.reshape(1, IN), lin1_W, lin1_b.reshape(1, IN),
        W1_rel, W1_root, b1.reshape(1, HID))
    zrows = jnp.zeros((CHUNK, HID), _f32)
    agg1_0, agg1_1, cnt0, cnt1 = _agg_cache(HID, True)(t1, srcp1, dst2, zrows)
    t2p, root2 = _m2(
        root1, agg1_0, agg1_1, cnt0, cnt1, W2_rel, W2_root, b2.reshape(1, OUT))
    agg2_0, agg2_1 = _agg_cache(2 * OUT, False)(t2p, srcp2, dst2, zrows)

    wuv = jnp.concatenate([fc_W[:OUT], fc_W[OUT:]], axis=1)
    buv = jnp.concatenate([fc_b, jnp.zeros((1,), _f32)]).reshape(1, 2)
    uv = _m3(root2, agg2_0, agg2_1, cnt0, cnt1, wuv, buv)
    out = _decode_cache()(uv[:, 0], uv[:, 1], i0p, i1p)
    return out[:Q].reshape(Q, 1)
